# baseline (speedup 1.0000x reference)
import functools

import jax
import jax.numpy as jnp
from jax import lax
from jax.experimental import pallas as pl
from jax.experimental.pallas import tpu as pltpu
from jax.experimental.pallas import tpu_sc as plsc


def _build(N, D, rows_per_worker, K):
    chunks = rows_per_worker // K
    nquads = chunks // 4
    mesh = plsc.VectorSubcoreMesh(core_axis_name="c", subcore_axis_name="s")
    nc = mesh.num_cores

    def body(x_hbm, idx_hbm, tab_hbm, out_hbm,
             idx_all, b0, b1, b2, b3, r0, r1,
             sg0, sg1, sx0, sx1, sx2, sx3, so0, so1, so2, so3):
        wid = lax.axis_index("s") * nc + lax.axis_index("c")
        base = wid * rows_per_worker

        bufs = (b0, b1, b2, b3)
        rbufs = (r0, r1)
        sxs = (sx0, sx1, sx2, sx3)
        sos = (so0, so1, so2, so3)

        pltpu.sync_copy(idx_hbm.at[pl.ds(base, rows_per_worker)], idx_all)

        def start_gather(j, r, sem):
            pltpu.async_copy(tab_hbm.at[idx_all.at[pl.ds(j * K, K)]], r, sem)

        def start_x(j, buf, sem):
            pltpu.async_copy(x_hbm.at[pl.ds(base + j * K, K)], buf, sem)

        def wait_into(buf, sem):
            pltpu.make_async_copy(x_hbm.at[pl.ds(0, K)], buf, sem).wait()

        def start_out(j, buf, sem):
            pltpu.async_copy(buf, out_hbm.at[pl.ds(base + j * K, K)], sem)

        def wait_out(j, buf, sem):
            pltpu.make_async_copy(buf, out_hbm.at[pl.ds(base + j * K, K)], sem).wait()

        def add_chunk(buf, rbuf):
            @plsc.parallel_loop(0, K, unroll=4)
            def add_row(r):
                for j in range(D // 16):
                    sl = pl.ds(j * 16, 16)
                    plsc.addupdate(buf.at[r, sl], rbuf[r, sl])

        start_gather(0, r0, sg0)
        start_x(0, b0, sx0)
        start_gather(1, r1, sg1)
        start_x(1, b1, sx1)
        start_x(2, b2, sx2)
        start_x(3, b3, sx3)

        def quad(q, _):
            j0 = 4 * q
            for k in range(4):
                j = j0 + k
                buf, sx, so = bufs[k], sxs[k], sos[k]
                rb, sg = rbufs[k % 2], (sg0, sg1)[k % 2]

                wait_into(rb, sg)
                wait_into(buf, sx)
                add_chunk(buf, rb)
                start_out(j, buf, so)

                @pl.when(j + 2 < chunks)
                def _():
                    start_gather(j + 2, rb, sg)

                @pl.when(j >= 2)
                def _():
                    pbuf = bufs[(k + 2) % 4]
                    wait_out(j - 2, pbuf, sos[(k + 2) % 4])

                    @pl.when(j + 2 < chunks)
                    def _():
                        start_x(j + 2, pbuf, sxs[(k + 2) % 4])

            return 0

        lax.fori_loop(0, nquads, quad, 0)

        wait_out(chunks - 2, bufs[(chunks - 2) % 4], sos[(chunks - 2) % 4])
        wait_out(chunks - 1, bufs[(chunks - 1) % 4], sos[(chunks - 1) % 4])

    return pl.kernel(
        body,
        out_type=jax.ShapeDtypeStruct((N, D), jnp.float32),
        mesh=mesh,
        scratch_types=[
            pltpu.VMEM((rows_per_worker,), jnp.int32),
            pltpu.VMEM((K, D), jnp.float32),
            pltpu.VMEM((K, D), jnp.float32),
            pltpu.VMEM((K, D), jnp.float32),
            pltpu.VMEM((K, D), jnp.float32),
            pltpu.VMEM((K, D), jnp.float32),
            pltpu.VMEM((K, D), jnp.float32),
        ] + [pltpu.SemaphoreType.DMA] * 10,
    )


@jax.jit
def kernel(x, positions, pos_emb):
    B, S, D = x.shape
    N = B * S
    nw = 32
    rows_per_worker = N // nw
    fn = _build(N, D, rows_per_worker, K=16)
    out = fn(x.reshape(N, D), positions.reshape(N), pos_emb)
    return out.reshape(B, S, D)


# speedup vs baseline: 1.2952x; 1.2952x over previous
import functools

import jax
import jax.numpy as jnp
from jax import lax
from jax.experimental import pallas as pl
from jax.experimental.pallas import tpu as pltpu
from jax.experimental.pallas import tpu_sc as plsc


def _build(N, D, rows_per_worker, K):
    chunks = rows_per_worker // K
    nquads = chunks // 4
    mesh = plsc.VectorSubcoreMesh(core_axis_name="c", subcore_axis_name="s")
    nc = mesh.num_cores

    def body(x_hbm, idx_hbm, tab_hbm, out_hbm,
             idx_all, b0, b1, b2, b3, r0, r1,
             sg0, sg1, sx0, sx1, sx2, sx3, so0, so1, so2, so3):
        wid = lax.axis_index("s") * nc + lax.axis_index("c")
        base = wid * rows_per_worker

        bufs = (b0, b1, b2, b3)
        rbufs = (r0, r1)
        sxs = (sx0, sx1, sx2, sx3)
        sos = (so0, so1, so2, so3)

        pltpu.sync_copy(idx_hbm.at[pl.ds(base, rows_per_worker)], idx_all)

        def start_gather(j, r, sem):
            pltpu.async_copy(tab_hbm.at[idx_all.at[pl.ds(j * K, K)]], r, sem)

        def start_x(j, buf, sem):
            pltpu.async_copy(x_hbm.at[pl.ds(base + j * K, K)], buf, sem)

        def wait_into(buf, sem):
            pltpu.make_async_copy(x_hbm.at[pl.ds(0, K)], buf, sem).wait()

        def start_out(j, buf, sem):
            pltpu.async_copy(buf, out_hbm.at[pl.ds(base + j * K, K)], sem)

        def wait_out(j, buf, sem):
            pltpu.make_async_copy(buf, out_hbm.at[pl.ds(base + j * K, K)], sem).wait()

        def add_chunk(buf, rbuf):
            @plsc.parallel_loop(0, K, unroll=1)
            def add_row(r):
                for j in range(D // 16):
                    sl = pl.ds(j * 16, 16)
                    plsc.addupdate(buf.at[r, sl], rbuf[r, sl])

        start_gather(0, r0, sg0)
        start_x(0, b0, sx0)
        start_gather(1, r1, sg1)
        start_x(1, b1, sx1)
        start_x(2, b2, sx2)
        start_x(3, b3, sx3)

        def quad(q, _):
            j0 = 4 * q
            for k in range(4):
                j = j0 + k
                buf, sx, so = bufs[k], sxs[k], sos[k]
                rb, sg = rbufs[k % 2], (sg0, sg1)[k % 2]

                wait_into(rb, sg)
                wait_into(buf, sx)
                add_chunk(buf, rb)
                start_out(j, buf, so)

                @pl.when(j + 2 < chunks)
                def _():
                    start_gather(j + 2, rb, sg)

                @pl.when(j >= 2)
                def _():
                    pbuf = bufs[(k + 2) % 4]
                    wait_out(j - 2, pbuf, sos[(k + 2) % 4])

                    @pl.when(j + 2 < chunks)
                    def _():
                        start_x(j + 2, pbuf, sxs[(k + 2) % 4])

            return 0

        lax.fori_loop(0, nquads, quad, 0)

        wait_out(chunks - 2, bufs[(chunks - 2) % 4], sos[(chunks - 2) % 4])
        wait_out(chunks - 1, bufs[(chunks - 1) % 4], sos[(chunks - 1) % 4])

    return pl.kernel(
        body,
        out_type=jax.ShapeDtypeStruct((N, D), jnp.float32),
        mesh=mesh,
        scratch_types=[
            pltpu.VMEM((rows_per_worker,), jnp.int32),
            pltpu.VMEM((K, D), jnp.float32),
            pltpu.VMEM((K, D), jnp.float32),
            pltpu.VMEM((K, D), jnp.float32),
            pltpu.VMEM((K, D), jnp.float32),
            pltpu.VMEM((K, D), jnp.float32),
            pltpu.VMEM((K, D), jnp.float32),
        ] + [pltpu.SemaphoreType.DMA] * 10,
    )


@jax.jit
def kernel(x, positions, pos_emb):
    B, S, D = x.shape
    N = B * S
    nw = 32
    rows_per_worker = N // nw
    fn = _build(N, D, rows_per_worker, K=16)
    out = fn(x.reshape(N, D), positions.reshape(N), pos_emb)
    return out.reshape(B, S, D)


# speedup vs baseline: 1.4151x; 1.0926x over previous
import functools

import jax
import jax.numpy as jnp
from jax import lax
from jax.experimental import pallas as pl
from jax.experimental.pallas import tpu as pltpu
from jax.experimental.pallas import tpu_sc as plsc


def _build(N, D, rows_per_worker, K):
    chunks = rows_per_worker // K
    nquads = chunks // 4
    mesh = plsc.VectorSubcoreMesh(core_axis_name="c", subcore_axis_name="s")
    nc = mesh.num_cores

    def body(x_hbm, idx_hbm, tab_hbm, out_hbm,
             idx_all, b0, b1, b2, b3, r0, r1,
             sg0, sg1, sx0, sx1, sx2, sx3, so0, so1, so2, so3):
        wid = lax.axis_index("s") * nc + lax.axis_index("c")
        base = wid * rows_per_worker

        bufs = (b0, b1, b2, b3)
        rbufs = (r0, r1)
        sxs = (sx0, sx1, sx2, sx3)
        sos = (so0, so1, so2, so3)

        pltpu.sync_copy(idx_hbm.at[pl.ds(base, rows_per_worker)], idx_all)

        def start_gather(j, r, sem):
            pltpu.async_copy(tab_hbm.at[idx_all.at[pl.ds(j * K, K)]], r, sem)

        def start_x(j, buf, sem):
            pltpu.async_copy(x_hbm.at[pl.ds(base + j * K, K)], buf, sem)

        def wait_into(buf, sem):
            pltpu.make_async_copy(x_hbm.at[pl.ds(0, K)], buf, sem).wait()

        def start_out(j, buf, sem):
            pltpu.async_copy(buf, out_hbm.at[pl.ds(base + j * K, K)], sem)

        def wait_out(j, buf, sem):
            pltpu.make_async_copy(buf, out_hbm.at[pl.ds(base + j * K, K)], sem).wait()

        def add_chunk(buf, rbuf):
            half = D // 32

            @plsc.parallel_loop(0, 2 * K, unroll=1)
            def add_half(h):
                r = h // 2
                off = (h % 2) * half
                for j in range(half // 16):
                    sl = pl.ds(off + j * 16, 16)
                    plsc.addupdate(buf.at[r, sl], rbuf[r, sl])

        start_gather(0, r0, sg0)
        start_x(0, b0, sx0)
        start_gather(1, r1, sg1)
        start_x(1, b1, sx1)
        start_x(2, b2, sx2)
        start_x(3, b3, sx3)

        def quad(q, _):
            j0 = 4 * q
            for k in range(4):
                j = j0 + k
                buf, sx, so = bufs[k], sxs[k], sos[k]
                rb, sg = rbufs[k % 2], (sg0, sg1)[k % 2]

                wait_into(rb, sg)
                wait_into(buf, sx)
                add_chunk(buf, rb)
                start_out(j, buf, so)

                @pl.when(j + 2 < chunks)
                def _():
                    start_gather(j + 2, rb, sg)

                @pl.when(j >= 2)
                def _():
                    pbuf = bufs[(k + 2) % 4]
                    wait_out(j - 2, pbuf, sos[(k + 2) % 4])

                    @pl.when(j + 2 < chunks)
                    def _():
                        start_x(j + 2, pbuf, sxs[(k + 2) % 4])

            return 0

        lax.fori_loop(0, nquads, quad, 0)

        wait_out(chunks - 2, bufs[(chunks - 2) % 4], sos[(chunks - 2) % 4])
        wait_out(chunks - 1, bufs[(chunks - 1) % 4], sos[(chunks - 1) % 4])

    return pl.kernel(
        body,
        out_type=jax.ShapeDtypeStruct((N, D), jnp.float32),
        mesh=mesh,
        scratch_types=[
            pltpu.VMEM((rows_per_worker,), jnp.int32),
            pltpu.VMEM((K, D), jnp.float32),
            pltpu.VMEM((K, D), jnp.float32),
            pltpu.VMEM((K, D), jnp.float32),
            pltpu.VMEM((K, D), jnp.float32),
            pltpu.VMEM((K, D), jnp.float32),
            pltpu.VMEM((K, D), jnp.float32),
        ] + [pltpu.SemaphoreType.DMA] * 10,
    )


@jax.jit
def kernel(x, positions, pos_emb):
    B, S, D = x.shape
    N = B * S
    nw = 32
    rows_per_worker = N // nw
    fn = _build(N, D, rows_per_worker, K=16)
    out = fn(x.reshape(N, D), positions.reshape(N), pos_emb)
    return out.reshape(B, S, D)
